# baseline (speedup 1.0000x reference)
import functools

import jax
import jax.numpy as jnp
from jax import lax
from jax.experimental import pallas as pl
from jax.experimental.pallas import tpu as pltpu
from jax.experimental.pallas import tpu_sc as plsc

N = 10000
NUM_POS = 640000
NUM_NEG = 640000
NE = NUM_POS + NUM_NEG

NC, NS, L = 2, 16, 16
NW = NC * NS
CH = NUM_POS // NS
NWIN = 20
WIN = CH // NWIN


def _softplus16(z):
    u = jnp.exp(-jnp.abs(z))
    s = u / (u + 2.0)
    t = s * s
    log1p = 2.0 * s * (1.0 + t * (1.0 / 3.0 + t * (0.2 + t * (1.0 / 7.0))))
    return jnp.maximum(z, 0.0) + log1p


def _sc_body(flat_ref, pos_ref, neg_ref, out_ref, rbuf, cbuf, vbuf, abuf, sem, isem):
    wid = lax.axis_index("s") * NC + lax.axis_index("c")
    half = wid // NS
    slot = wid % NS
    base = slot * CH

    @pl.when(half == 0)
    def _():
        c1 = pltpu.async_copy(pos_ref.at[pl.ds(base, CH)], rbuf, isem)
        c2 = pltpu.async_copy(pos_ref.at[pl.ds(NUM_POS + base, CH)], cbuf, isem)
        c1.wait()
        c2.wait()

    @pl.when(half == 1)
    def _():
        c1 = pltpu.async_copy(neg_ref.at[pl.ds(base, CH)], rbuf, isem)
        c2 = pltpu.async_copy(neg_ref.at[pl.ds(NUM_NEG + base, CH)], cbuf, isem)
        c1.wait()
        c2.wait()

    sign = jnp.where(half == 0, -1.0, 1.0)

    gathers = []
    for j in range(NWIN):
        o = j * WIN

        @plsc.parallel_loop(o, o + WIN, step=L, unroll=5)
        def _flat(i):
            rbuf[pl.ds(i, L)] = rbuf[pl.ds(i, L)] * N + cbuf[pl.ds(i, L)]

        gathers.append(
            pltpu.async_copy(
                flat_ref.at[rbuf.at[pl.ds(o, WIN)]], vbuf.at[pl.ds(o, WIN)], sem))

    acc = jnp.zeros((L,), jnp.float32)
    for j, g in enumerate(gathers):
        o = j * WIN
        g.wait()

        @plsc.parallel_loop(o, o + WIN, step=L, unroll=5, carry=acc)
        def _acc(i, a):
            z = vbuf[pl.ds(i, L)] * sign
            return a + _softplus16(z)

        acc = _acc

    abuf[...] = acc
    pltpu.sync_copy(abuf, out_ref.at[pl.ds(wid * L, L)])


@functools.partial(
    pl.kernel,
    out_type=jax.ShapeDtypeStruct((NW * L,), jnp.float32),
    mesh=plsc.VectorSubcoreMesh(core_axis_name="c", subcore_axis_name="s"),
    scratch_types=[
        pltpu.VMEM((CH,), jnp.int32),
        pltpu.VMEM((CH,), jnp.int32),
        pltpu.VMEM((CH,), jnp.float32),
        pltpu.VMEM((L,), jnp.float32),
        pltpu.SemaphoreType.DMA,
        pltpu.SemaphoreType.DMA,
    ],
)
def _sc_loss(flat_ref, pos_ref, neg_ref, out_ref, rbuf, cbuf, vbuf, abuf, sem, isem):
    _sc_body(flat_ref, pos_ref, neg_ref, out_ref, rbuf, cbuf, vbuf, abuf, sem, isem)


def kernel(input, pos_edge_index, neg_edge_index):
    flat = input.reshape(-1)
    pos = pos_edge_index.astype(jnp.int32).reshape(-1)
    neg = neg_edge_index.astype(jnp.int32).reshape(-1)
    partials = _sc_loss(flat, pos, neg)
    return jnp.sum(partials) / NE
